# baseline (speedup 1.0000x reference)
import functools

import jax
import jax.numpy as jnp
from jax import lax
from jax.experimental import pallas as pl
from jax.experimental.pallas import tpu as pltpu
from jax.experimental.pallas import tpu_sc as plsc

_N = 10000
_E = 320000
_FIN = 128
_H = 16
_C = 40

_NC = 2
_NS = 16
_NW = _NC * _NS
_CHUNK = 128
_EPT_CH = 80
_EPAD = _NW * _EPT_CH * _CHUNK
_NBUF = 8
_NPAD = 10240
_TRASH = 10016
_RPT = _NPAD // _NS
_D2 = 48

@functools.cache
def _mesh():
    return plsc.VectorSubcoreMesh(core_axis_name="core",
                                  subcore_axis_name="subcore")


_sc_params = pltpu.CompilerParams(use_tc_tiling_on_sc=False)
_sc_params_nolayout = pltpu.CompilerParams(use_tc_tiling_on_sc=False,
                                           needs_layout_passes=False)


def _deg_body(dst_hbm, ones_hbm, zeros_hbm, out_hbm, idx_v, ones_v, dbuf,
              dbuf16, acc_sh):
    c = lax.axis_index("core")
    s = lax.axis_index("subcore")
    wid = c * _NS + s
    pltpu.sync_copy(zeros_hbm.at[pl.ds(s * _RPT, _RPT)],
                    acc_sh.at[pl.ds(s * _RPT, _RPT)])
    pltpu.sync_copy(ones_hbm, ones_v)
    pltpu.sync_copy(dst_hbm.at[wid], idx_v)
    plsc.subcore_barrier()

    @pl.loop(0, _EPT_CH)
    def _(j):
        pltpu.sync_copy(ones_v, acc_sh.at[idx_v.at[j]], add=True)

    plsc.subcore_barrier()
    pltpu.sync_copy(acc_sh.at[pl.ds(s * _RPT, _RPT)], dbuf)

    @pl.loop(0, _RPT // 16)
    def _(q):
        v = dbuf[pl.ds(q * 16, 16)]
        rowi = q * 16 + lax.iota(jnp.int32, 16)
        coli = jnp.zeros((16,), jnp.int32)
        plsc.store_scatter(dbuf16, [rowi, coli], v)

    pltpu.sync_copy(dbuf16, out_hbm.at[c, pl.ds(s * _RPT, _RPT), pl.ds(0, 16)])


def _deg_pass(dstp, ones, zeros1):
    return pl.kernel(
        _deg_body,
        out_type=jax.ShapeDtypeStruct((_NC, _NPAD, 128), jnp.float32),
        mesh=_mesh(),
        scratch_types=[
            pltpu.VMEM((_EPT_CH, _CHUNK), jnp.int32),
            pltpu.VMEM((_CHUNK,), jnp.float32),
            pltpu.VMEM((_RPT,), jnp.float32),
            pltpu.VMEM((_RPT, 16), jnp.float32),
            pltpu.VMEM_SHARED((_NPAD,), jnp.float32),
        ],
        compiler_params=_sc_params_nolayout,
    )(dstp, ones, zeros1)


def _edge_body(d, src_hbm, dst_hbm, gp_hbm, zeros_hbm, gtab_hbm, out_hbm,
               sidx, didx, rows, gbuf, acc_sh, *sems):
    c = lax.axis_index("core")
    s = lax.axis_index("subcore")
    wid = c * _NS + s
    pltpu.sync_copy(zeros_hbm.at[pl.ds(s * _RPT, _RPT), :],
                    acc_sh.at[pl.ds(s * _RPT, _RPT), :])
    pltpu.sync_copy(gp_hbm.at[pl.ds(s * _RPT, _RPT), pl.ds(0, d)], gbuf)
    pltpu.sync_copy(gbuf, gtab_hbm.at[c, pl.ds(s * _RPT, _RPT), :])
    pltpu.sync_copy(src_hbm.at[wid], sidx)
    pltpu.sync_copy(dst_hbm.at[wid], didx)
    plsc.subcore_barrier()
    gtab = gtab_hbm.at[c]

    def g_start(j, k):
        pltpu.async_copy(gtab.at[sidx.at[j]], rows.at[k], sems[k])

    def g_wait(k):
        pltpu.make_async_copy(gtab.at[sidx.at[0]], rows.at[k], sems[k]).wait()

    def s_start(j, k):
        pltpu.async_copy(rows.at[k], acc_sh.at[didx.at[j]], sems[k], add=True)

    def s_wait(k):
        pltpu.make_async_copy(rows.at[k], acc_sh.at[didx.at[0]],
                              sems[k]).wait()

    for k in range(_NBUF):
        g_start(k, k)

    @pl.loop(0, _EPT_CH // _NBUF - 1)
    def _(q):
        j = q * _NBUF
        for k in range(_NBUF):
            g_wait(k)
            s_start(j + k, k)
        for k in range(_NBUF):
            s_wait(k)
            g_start(j + _NBUF + k, k)

    jl = _EPT_CH - _NBUF
    for k in range(_NBUF):
        g_wait(k)
        s_start(jl + k, k)
    for k in range(_NBUF):
        s_wait(k)

    plsc.subcore_barrier()
    pltpu.sync_copy(acc_sh.at[pl.ds(s * _RPT, _RPT), :],
                    out_hbm.at[c, pl.ds(s * _RPT, _RPT), pl.ds(0, d)])


def _edge_pass(d, srcp, dstp, gp, zeros):
    _, accb = pl.kernel(
        functools.partial(_edge_body, d),
        out_type=[
            jax.ShapeDtypeStruct((_NC, _NPAD, d), jnp.float32),
            jax.ShapeDtypeStruct((_NC, _NPAD, 128), jnp.float32),
        ],
        mesh=_mesh(),
        scratch_types=[
            pltpu.VMEM((_EPT_CH, _CHUNK), jnp.int32),
            pltpu.VMEM((_EPT_CH, _CHUNK), jnp.int32),
            pltpu.VMEM((_NBUF, _CHUNK, d), jnp.float32),
            pltpu.VMEM((_RPT, d), jnp.float32),
            pltpu.VMEM_SHARED((_NPAD, d), jnp.float32),
        ] + [pltpu.SemaphoreType.DMA] * _NBUF,
        compiler_params=_sc_params,
    )(srcp, dstp, gp, zeros)
    return accb


def _b1a_body(x_ref, w1_ref, h1p_ref):
    h1p_ref[...] = jnp.dot(x_ref[...], w1_ref[...],
                           preferred_element_type=jnp.float32)


def _b1b_body(degb_ref, h1p_ref, g1p_ref, dis_ref):
    deg = degb_ref[0, :, 0:1] + degb_ref[1, :, 0:1] + 1.0
    dis = lax.rsqrt(deg)
    g1p_ref[pl.ds(0, _N), :] = h1p_ref[...] * dis[0:_N, :]
    g1p_ref[pl.ds(_N, _NPAD - _N), :] = jnp.zeros((_NPAD - _N, 128),
                                                  jnp.float32)
    dis_ref[...] = dis[0:_N, :]


def _b2_body(acc_ref, g1_ref, dis_ref, w2_ref, b1_ref, g2p_ref):
    dis = dis_ref[...]
    out1 = ((acc_ref[0, pl.ds(0, _N), 0:_H] + acc_ref[1, pl.ds(0, _N), 0:_H]
             + g1_ref[pl.ds(0, _N), 0:_H]) * dis + b1_ref[...])
    h2p = jnp.dot(out1, w2_ref[...], preferred_element_type=jnp.float32)
    g2p_ref[pl.ds(0, _N), :] = h2p * dis
    g2p_ref[pl.ds(_N, _NPAD - _N), :] = jnp.zeros((_NPAD - _N, 128),
                                                  jnp.float32)


def _b3_body(acc_ref, g2_ref, dis_ref, b2_ref, out_ref):
    o48 = ((acc_ref[0, pl.ds(0, _N), 0:_D2] + acc_ref[1, pl.ds(0, _N), 0:_D2]
            + g2_ref[pl.ds(0, _N), 0:_D2]) * dis_ref[...] + b2_ref[...])
    out_ref[...] = o48[:, 0:_C]


def kernel(x, edge_index, W1, b1, W2, b2):
    src, dst = edge_index[0], edge_index[1]
    pad = _TRASH + (jnp.arange(_EPAD - _E, dtype=jnp.int32) % _CHUNK)
    srcp = jnp.concatenate([src, pad]).reshape(_NW, _EPT_CH, _CHUNK)
    dstp = jnp.concatenate([dst, pad]).reshape(_NW, _EPT_CH, _CHUNK)
    dstp = lax.optimization_barrier(dstp)
    srcp = lax.optimization_barrier(srcp)
    ones = jnp.ones((_CHUNK,), jnp.float32)
    zeros1 = jnp.zeros((_NPAD,), jnp.float32)
    zeros_h = jnp.zeros((_NPAD, _H), jnp.float32)
    zeros_d2 = jnp.zeros((_NPAD, _D2), jnp.float32)
    w1p = jnp.pad(W1, ((0, 0), (0, 128 - _H)))
    w2p = jnp.pad(W2, ((0, 0), (0, 128 - _C)))
    b1r = b1.reshape(1, _H)
    b2p = jnp.pad(b2, (0, _D2 - _C)).reshape(1, _D2)

    h1p = pl.pallas_call(
        _b1a_body,
        out_shape=jax.ShapeDtypeStruct((_N, 128), jnp.float32),
    )(x, w1p)

    degb = _deg_pass(dstp, ones, zeros1)

    g1p, dis = pl.pallas_call(
        _b1b_body,
        out_shape=[
            jax.ShapeDtypeStruct((_NPAD, 128), jnp.float32),
            jax.ShapeDtypeStruct((_N, 1), jnp.float32),
        ],
    )(degb, h1p)

    acc1 = _edge_pass(_H, srcp, dstp, g1p, zeros_h)

    g2p = pl.pallas_call(
        _b2_body,
        out_shape=jax.ShapeDtypeStruct((_NPAD, 128), jnp.float32),
    )(acc1, g1p, dis, w2p, b1r)

    acc2 = _edge_pass(_D2, srcp, dstp, g2p, zeros_d2)

    return pl.pallas_call(
        _b3_body,
        out_shape=jax.ShapeDtypeStruct((_N, _C), jnp.float32),
    )(acc2, g2p, dis, b2p)


# speedup vs baseline: 1.0396x; 1.0396x over previous
import functools

import jax
import jax.numpy as jnp
from jax import lax
from jax.experimental import pallas as pl
from jax.experimental.pallas import tpu as pltpu
from jax.experimental.pallas import tpu_sc as plsc

_N = 10000
_E = 320000
_FIN = 128
_H = 16
_C = 40

_NC = 2
_NS = 16
_NW = _NC * _NS
_CHUNK = 128
_EPT_CH = 80
_EPAD = _NW * _EPT_CH * _CHUNK
_NBUF = 8
_NPAD = 10240
_TRASH = 10016
_RPT = _NPAD // _NS
_D2 = 40

@functools.cache
def _mesh():
    return plsc.VectorSubcoreMesh(core_axis_name="core",
                                  subcore_axis_name="subcore")


_sc_params = pltpu.CompilerParams(use_tc_tiling_on_sc=False)
_sc_params_nolayout = pltpu.CompilerParams(use_tc_tiling_on_sc=False,
                                           needs_layout_passes=False)


def _deg_body(dst_hbm, ones_hbm, zeros_hbm, out_hbm, idx_v, ones_v, dbuf,
              dbuf16, acc_sh):
    c = lax.axis_index("core")
    s = lax.axis_index("subcore")
    wid = c * _NS + s
    pltpu.sync_copy(zeros_hbm.at[pl.ds(s * _RPT, _RPT)],
                    acc_sh.at[pl.ds(s * _RPT, _RPT)])
    pltpu.sync_copy(ones_hbm, ones_v)
    pltpu.sync_copy(dst_hbm.at[wid], idx_v)
    plsc.subcore_barrier()

    @pl.loop(0, _EPT_CH)
    def _(j):
        pltpu.sync_copy(ones_v, acc_sh.at[idx_v.at[j]], add=True)

    plsc.subcore_barrier()
    pltpu.sync_copy(acc_sh.at[pl.ds(s * _RPT, _RPT)], dbuf)

    @pl.loop(0, _RPT // 16)
    def _(q):
        v = dbuf[pl.ds(q * 16, 16)]
        rowi = q * 16 + lax.iota(jnp.int32, 16)
        coli = jnp.zeros((16,), jnp.int32)
        plsc.store_scatter(dbuf16, [rowi, coli], v)

    pltpu.sync_copy(dbuf16, out_hbm.at[c, pl.ds(s * _RPT, _RPT), pl.ds(0, 16)])


def _deg_pass(dstp, ones, zeros1):
    return pl.kernel(
        _deg_body,
        out_type=jax.ShapeDtypeStruct((_NC, _NPAD, 128), jnp.float32),
        mesh=_mesh(),
        scratch_types=[
            pltpu.VMEM((_EPT_CH, _CHUNK), jnp.int32),
            pltpu.VMEM((_CHUNK,), jnp.float32),
            pltpu.VMEM((_RPT,), jnp.float32),
            pltpu.VMEM((_RPT, 16), jnp.float32),
            pltpu.VMEM_SHARED((_NPAD,), jnp.float32),
        ],
        compiler_params=_sc_params_nolayout,
    )(dstp, ones, zeros1)


def _edge_body(d, src_hbm, dst_hbm, gp_hbm, zeros_hbm, gtab_hbm, out_hbm,
               sidx, didx, rows, gbuf, acc_sh, *sems):
    c = lax.axis_index("core")
    s = lax.axis_index("subcore")
    wid = c * _NS + s
    pltpu.sync_copy(zeros_hbm.at[pl.ds(s * _RPT, _RPT), :],
                    acc_sh.at[pl.ds(s * _RPT, _RPT), :])
    pltpu.sync_copy(gp_hbm.at[pl.ds(s * _RPT, _RPT), pl.ds(0, d)], gbuf)
    pltpu.sync_copy(gbuf, gtab_hbm.at[c, pl.ds(s * _RPT, _RPT), :])
    pltpu.sync_copy(src_hbm.at[wid], sidx)
    pltpu.sync_copy(dst_hbm.at[wid], didx)
    plsc.subcore_barrier()
    gtab = gtab_hbm.at[c]

    def g_start(j, k):
        pltpu.async_copy(gtab.at[sidx.at[j]], rows.at[k], sems[k])

    def g_wait(k):
        pltpu.make_async_copy(gtab.at[sidx.at[0]], rows.at[k], sems[k]).wait()

    def s_start(j, k):
        pltpu.async_copy(rows.at[k], acc_sh.at[didx.at[j]], sems[k], add=True)

    def s_wait(k):
        pltpu.make_async_copy(rows.at[k], acc_sh.at[didx.at[0]],
                              sems[k]).wait()

    for k in range(_NBUF):
        g_start(k, k)

    @pl.loop(0, _EPT_CH // _NBUF - 1)
    def _(q):
        j = q * _NBUF
        for k in range(_NBUF):
            g_wait(k)
            s_start(j + k, k)
        for k in range(_NBUF):
            s_wait(k)
            g_start(j + _NBUF + k, k)

    jl = _EPT_CH - _NBUF
    for k in range(_NBUF):
        g_wait(k)
        s_start(jl + k, k)
    for k in range(_NBUF):
        s_wait(k)

    plsc.subcore_barrier()
    pltpu.sync_copy(acc_sh.at[pl.ds(s * _RPT, _RPT), :],
                    out_hbm.at[c, pl.ds(s * _RPT, _RPT), pl.ds(0, d)])


def _edge_pass(d, srcp, dstp, gp, zeros):
    _, accb = pl.kernel(
        functools.partial(_edge_body, d),
        out_type=[
            jax.ShapeDtypeStruct((_NC, _NPAD, d), jnp.float32),
            jax.ShapeDtypeStruct((_NC, _NPAD, 128), jnp.float32),
        ],
        mesh=_mesh(),
        scratch_types=[
            pltpu.VMEM((_EPT_CH, _CHUNK), jnp.int32),
            pltpu.VMEM((_EPT_CH, _CHUNK), jnp.int32),
            pltpu.VMEM((_NBUF, _CHUNK, d), jnp.float32),
            pltpu.VMEM((_RPT, d), jnp.float32),
            pltpu.VMEM_SHARED((_NPAD, d), jnp.float32),
        ] + [pltpu.SemaphoreType.DMA] * _NBUF,
        compiler_params=_sc_params,
    )(srcp, dstp, gp, zeros)
    return accb


def _b1_body(degb_ref, x_ref, w1_ref, g1p_ref, dis_ref):
    deg = degb_ref[0, :, 0:1] + degb_ref[1, :, 0:1] + 1.0
    dis = lax.rsqrt(deg)
    hp = jnp.dot(x_ref[...], w1_ref[...], preferred_element_type=jnp.float32)
    g1p_ref[pl.ds(0, _N), :] = hp * dis[0:_N, :]
    g1p_ref[pl.ds(_N, _NPAD - _N), :] = jnp.zeros((_NPAD - _N, 128),
                                                  jnp.float32)
    dis_ref[...] = dis[0:_N, :]


def _b2_body(acc_ref, g1_ref, dis_ref, w2_ref, b1_ref, g2p_ref):
    dis = dis_ref[...]
    out1 = ((acc_ref[0, pl.ds(0, _N), 0:_H] + acc_ref[1, pl.ds(0, _N), 0:_H]
             + g1_ref[pl.ds(0, _N), 0:_H]) * dis + b1_ref[...])
    h2p = jnp.dot(out1, w2_ref[...], preferred_element_type=jnp.float32)
    g2p_ref[pl.ds(0, _N), :] = h2p * dis
    g2p_ref[pl.ds(_N, _NPAD - _N), :] = jnp.zeros((_NPAD - _N, 128),
                                                  jnp.float32)


def _b3_body(acc_ref, g2_ref, dis_ref, b2_ref, out_ref):
    o48 = ((acc_ref[0, pl.ds(0, _N), 0:_D2] + acc_ref[1, pl.ds(0, _N), 0:_D2]
            + g2_ref[pl.ds(0, _N), 0:_D2]) * dis_ref[...] + b2_ref[...])
    out_ref[...] = o48[:, 0:_C]


def kernel(x, edge_index, W1, b1, W2, b2):
    src, dst = edge_index[0], edge_index[1]
    pad = _TRASH + (jnp.arange(_EPAD - _E, dtype=jnp.int32) % _CHUNK)
    srcp = jnp.concatenate([src, pad]).reshape(_NW, _EPT_CH, _CHUNK)
    dstp = jnp.concatenate([dst, pad]).reshape(_NW, _EPT_CH, _CHUNK)
    dstp = lax.optimization_barrier(dstp)
    srcp = lax.optimization_barrier(srcp)
    ones = jnp.ones((_CHUNK,), jnp.float32)
    zeros1 = jnp.zeros((_NPAD,), jnp.float32)
    zeros_h = jnp.zeros((_NPAD, _H), jnp.float32)
    zeros_d2 = jnp.zeros((_NPAD, _D2), jnp.float32)
    w1p = jnp.pad(W1, ((0, 0), (0, 128 - _H)))
    w2p = jnp.pad(W2, ((0, 0), (0, 128 - _C)))
    b1r = b1.reshape(1, _H)
    b2p = jnp.pad(b2, (0, _D2 - _C)).reshape(1, _D2)

    degb = _deg_pass(dstp, ones, zeros1)

    g1p, dis = pl.pallas_call(
        _b1_body,
        out_shape=[
            jax.ShapeDtypeStruct((_NPAD, 128), jnp.float32),
            jax.ShapeDtypeStruct((_N, 1), jnp.float32),
        ],
    )(degb, x, w1p)

    acc1 = _edge_pass(_H, srcp, dstp, g1p, zeros_h)

    g2p = pl.pallas_call(
        _b2_body,
        out_shape=jax.ShapeDtypeStruct((_NPAD, 128), jnp.float32),
    )(acc1, g1p, dis, w2p, b1r)

    acc2 = _edge_pass(_D2, srcp, dstp, g2p, zeros_d2)

    return pl.pallas_call(
        _b3_body,
        out_shape=jax.ShapeDtypeStruct((_N, _C), jnp.float32),
    )(acc2, g2p, dis, b2p)


# speedup vs baseline: 1.0739x; 1.0330x over previous
import functools

import jax
import jax.numpy as jnp
from jax import lax
from jax.experimental import pallas as pl
from jax.experimental.pallas import tpu as pltpu
from jax.experimental.pallas import tpu_sc as plsc

_N = 10000
_E = 320000
_FIN = 128
_H = 16
_C = 40

_NC = 2
_NS = 16
_NW = _NC * _NS
_CHUNK = 128
_EPT_CH = 80
_EPAD = _NW * _EPT_CH * _CHUNK
_NBUF = 8
_NPAD = 10240
_TRASH = 10016
_RPT = _NPAD // _NS
_D2 = 40

@functools.cache
def _mesh():
    return plsc.VectorSubcoreMesh(core_axis_name="core",
                                  subcore_axis_name="subcore")


_sc_params = pltpu.CompilerParams(use_tc_tiling_on_sc=False)
_sc_params_nolayout = pltpu.CompilerParams(use_tc_tiling_on_sc=False,
                                           needs_layout_passes=False)


def _deg_body(dst_hbm, ones_hbm, zeros_hbm, out_hbm, idx_v, ones_v, dbuf,
              dbuf16, acc_sh, dsem):
    c = lax.axis_index("core")
    s = lax.axis_index("subcore")
    wid = c * _NS + s
    pltpu.sync_copy(zeros_hbm.at[pl.ds(s * _RPT, _RPT)],
                    acc_sh.at[pl.ds(s * _RPT, _RPT)])
    pltpu.sync_copy(ones_hbm, ones_v)
    pltpu.sync_copy(dst_hbm.at[wid], idx_v)
    plsc.subcore_barrier()

    @pl.loop(0, _EPT_CH)
    def _(j):
        pltpu.async_copy(ones_v, acc_sh.at[idx_v.at[j]], dsem, add=True)

    @pl.loop(0, _EPT_CH)
    def _(j):
        pltpu.make_async_copy(ones_v, acc_sh.at[idx_v.at[0]], dsem).wait()

    plsc.subcore_barrier()
    pltpu.sync_copy(acc_sh.at[pl.ds(s * _RPT, _RPT)], dbuf)

    @pl.loop(0, _RPT // 16)
    def _(q):
        v = dbuf[pl.ds(q * 16, 16)]
        rowi = q * 16 + lax.iota(jnp.int32, 16)
        coli = jnp.zeros((16,), jnp.int32)
        plsc.store_scatter(dbuf16, [rowi, coli], v)

    pltpu.sync_copy(dbuf16, out_hbm.at[c, pl.ds(s * _RPT, _RPT), pl.ds(0, 16)])


def _deg_pass(dstp, ones, zeros1):
    return pl.kernel(
        _deg_body,
        out_type=jax.ShapeDtypeStruct((_NC, _NPAD, 128), jnp.float32),
        mesh=_mesh(),
        scratch_types=[
            pltpu.VMEM((_EPT_CH, _CHUNK), jnp.int32),
            pltpu.VMEM((_CHUNK,), jnp.float32),
            pltpu.VMEM((_RPT,), jnp.float32),
            pltpu.VMEM((_RPT, 16), jnp.float32),
            pltpu.VMEM_SHARED((_NPAD,), jnp.float32),
            pltpu.SemaphoreType.DMA,
        ],
        compiler_params=_sc_params_nolayout,
    )(dstp, ones, zeros1)


def _edge_body(d, src_hbm, dst_hbm, gp_hbm, zeros_hbm, gtab_hbm, out_hbm,
               sidx, didx, rows, gbuf, acc_sh, *sems):
    c = lax.axis_index("core")
    s = lax.axis_index("subcore")
    wid = c * _NS + s
    pltpu.sync_copy(zeros_hbm.at[pl.ds(s * _RPT, _RPT), :],
                    acc_sh.at[pl.ds(s * _RPT, _RPT), :])
    pltpu.sync_copy(gp_hbm.at[pl.ds(s * _RPT, _RPT), pl.ds(0, d)], gbuf)
    pltpu.sync_copy(gbuf, gtab_hbm.at[c, pl.ds(s * _RPT, _RPT), :])
    pltpu.sync_copy(src_hbm.at[wid], sidx)
    pltpu.sync_copy(dst_hbm.at[wid], didx)
    plsc.subcore_barrier()
    gtab = gtab_hbm.at[c]

    def g_start(j, k):
        pltpu.async_copy(gtab.at[sidx.at[j]], rows.at[k], sems[k])

    def g_wait(k):
        pltpu.make_async_copy(gtab.at[sidx.at[0]], rows.at[k], sems[k]).wait()

    def s_start(j, k):
        pltpu.async_copy(rows.at[k], acc_sh.at[didx.at[j]], sems[k], add=True)

    def s_wait(k):
        pltpu.make_async_copy(rows.at[k], acc_sh.at[didx.at[0]],
                              sems[k]).wait()

    for k in range(_NBUF):
        g_start(k, k)

    @pl.loop(0, _EPT_CH // _NBUF - 1)
    def _(q):
        j = q * _NBUF
        for k in range(_NBUF):
            g_wait(k)
            s_start(j + k, k)
        for k in range(_NBUF):
            s_wait(k)
            g_start(j + _NBUF + k, k)

    jl = _EPT_CH - _NBUF
    for k in range(_NBUF):
        g_wait(k)
        s_start(jl + k, k)
    for k in range(_NBUF):
        s_wait(k)

    plsc.subcore_barrier()
    pltpu.sync_copy(acc_sh.at[pl.ds(s * _RPT, _RPT), :],
                    out_hbm.at[c, pl.ds(s * _RPT, _RPT), pl.ds(0, d)])


def _edge_pass(d, srcp, dstp, gp, zeros):
    _, accb = pl.kernel(
        functools.partial(_edge_body, d),
        out_type=[
            jax.ShapeDtypeStruct((_NC, _NPAD, d), jnp.float32),
            jax.ShapeDtypeStruct((_NC, _NPAD, 128), jnp.float32),
        ],
        mesh=_mesh(),
        scratch_types=[
            pltpu.VMEM((_EPT_CH, _CHUNK), jnp.int32),
            pltpu.VMEM((_EPT_CH, _CHUNK), jnp.int32),
            pltpu.VMEM((_NBUF, _CHUNK, d), jnp.float32),
            pltpu.VMEM((_RPT, d), jnp.float32),
            pltpu.VMEM_SHARED((_NPAD, d), jnp.float32),
        ] + [pltpu.SemaphoreType.DMA] * _NBUF,
        compiler_params=_sc_params,
    )(srcp, dstp, gp, zeros)
    return accb


def _b1_body(degb_ref, x_ref, w1_ref, g1p_ref, dis_ref):
    deg = degb_ref[0, :, 0:1] + degb_ref[1, :, 0:1] + 1.0
    dis = lax.rsqrt(deg)
    hp = jnp.dot(x_ref[...], w1_ref[...], preferred_element_type=jnp.float32)
    g1p_ref[pl.ds(0, _N), :] = hp * dis[0:_N, :]
    g1p_ref[pl.ds(_N, _NPAD - _N), :] = jnp.zeros((_NPAD - _N, 128),
                                                  jnp.float32)
    dis_ref[...] = dis[0:_N, :]


def _b2_body(acc_ref, g1_ref, dis_ref, w2_ref, b1_ref, g2p_ref):
    dis = dis_ref[...]
    out1 = ((acc_ref[0, pl.ds(0, _N), 0:_H] + acc_ref[1, pl.ds(0, _N), 0:_H]
             + g1_ref[pl.ds(0, _N), 0:_H]) * dis + b1_ref[...])
    h2p = jnp.dot(out1, w2_ref[...], preferred_element_type=jnp.float32)
    g2p_ref[pl.ds(0, _N), :] = h2p * dis
    g2p_ref[pl.ds(_N, _NPAD - _N), :] = jnp.zeros((_NPAD - _N, 128),
                                                  jnp.float32)


def _b3_body(acc_ref, g2_ref, dis_ref, b2_ref, out_ref):
    o48 = ((acc_ref[0, pl.ds(0, _N), 0:_D2] + acc_ref[1, pl.ds(0, _N), 0:_D2]
            + g2_ref[pl.ds(0, _N), 0:_D2]) * dis_ref[...] + b2_ref[...])
    out_ref[...] = o48[:, 0:_C]


def kernel(x, edge_index, W1, b1, W2, b2):
    src, dst = edge_index[0], edge_index[1]
    pad = _TRASH + (jnp.arange(_EPAD - _E, dtype=jnp.int32) % _CHUNK)
    srcp = jnp.concatenate([src, pad]).reshape(_NW, _EPT_CH, _CHUNK)
    dstp = jnp.concatenate([dst, pad]).reshape(_NW, _EPT_CH, _CHUNK)
    dstp = lax.optimization_barrier(dstp)
    srcp = lax.optimization_barrier(srcp)
    ones = jnp.ones((_CHUNK,), jnp.float32)
    zeros1 = jnp.zeros((_NPAD,), jnp.float32)
    zeros_h = jnp.zeros((_NPAD, _H), jnp.float32)
    zeros_d2 = jnp.zeros((_NPAD, _D2), jnp.float32)
    w1p = jnp.pad(W1, ((0, 0), (0, 128 - _H)))
    w2p = jnp.pad(W2, ((0, 0), (0, 128 - _C)))
    b1r = b1.reshape(1, _H)
    b2p = jnp.pad(b2, (0, _D2 - _C)).reshape(1, _D2)

    degb = _deg_pass(dstp, ones, zeros1)

    g1p, dis = pl.pallas_call(
        _b1_body,
        out_shape=[
            jax.ShapeDtypeStruct((_NPAD, 128), jnp.float32),
            jax.ShapeDtypeStruct((_N, 1), jnp.float32),
        ],
    )(degb, x, w1p)

    acc1 = _edge_pass(_H, srcp, dstp, g1p, zeros_h)

    g2p = pl.pallas_call(
        _b2_body,
        out_shape=jax.ShapeDtypeStruct((_NPAD, 128), jnp.float32),
    )(acc1, g1p, dis, w2p, b1r)

    acc2 = _edge_pass(_D2, srcp, dstp, g2p, zeros_d2)

    return pl.pallas_call(
        _b3_body,
        out_shape=jax.ShapeDtypeStruct((_N, _C), jnp.float32),
    )(acc2, g2p, dis, b2p)


# speedup vs baseline: 1.0908x; 1.0158x over previous
import functools

import jax
import jax.numpy as jnp
from jax import lax
from jax.experimental import pallas as pl
from jax.experimental.pallas import tpu as pltpu
from jax.experimental.pallas import tpu_sc as plsc

_N = 10000
_E = 320000
_FIN = 128
_H = 16
_C = 40

_NC = 2
_NS = 16
_NW = _NC * _NS
_CHUNK = 128
_EPT_CH = 80
_EPAD = _NW * _EPT_CH * _CHUNK
_NBUF = 8
_NPAD = 10240
_TRASH = 10016
_RPT = _NPAD // _NS
_D2 = 40

@functools.cache
def _mesh():
    return plsc.VectorSubcoreMesh(core_axis_name="core",
                                  subcore_axis_name="subcore")


_sc_params = pltpu.CompilerParams(use_tc_tiling_on_sc=False)
_sc_params_nolayout = pltpu.CompilerParams(use_tc_tiling_on_sc=False,
                                           needs_layout_passes=False)


def _deg_body(dst_hbm, ones_hbm, zeros_hbm, out_hbm, idx_v, ones_v, dbuf,
              dbuf16, acc_sh, dsem):
    c = lax.axis_index("core")
    s = lax.axis_index("subcore")
    wid = c * _NS + s
    pltpu.sync_copy(zeros_hbm.at[pl.ds(s * _RPT, _RPT)],
                    acc_sh.at[pl.ds(s * _RPT, _RPT)])
    pltpu.sync_copy(ones_hbm, ones_v)
    pltpu.sync_copy(dst_hbm.at[wid], idx_v)
    plsc.subcore_barrier()

    @pl.loop(0, _EPT_CH)
    def _(j):
        pltpu.async_copy(ones_v, acc_sh.at[idx_v.at[j]], dsem, add=True)

    @pl.loop(0, _EPT_CH)
    def _(j):
        pltpu.make_async_copy(ones_v, acc_sh.at[idx_v.at[0]], dsem).wait()

    plsc.subcore_barrier()
    pltpu.sync_copy(acc_sh.at[pl.ds(s * _RPT, _RPT)], dbuf)

    @pl.loop(0, _RPT // 16)
    def _(q):
        v = dbuf[pl.ds(q * 16, 16)]
        rowi = q * 16 + lax.iota(jnp.int32, 16)
        coli = jnp.zeros((16,), jnp.int32)
        plsc.store_scatter(dbuf16, [rowi, coli], v)

    pltpu.sync_copy(dbuf16,
                    out_hbm.at[pl.ds(s * _RPT, _RPT), pl.ds(c * 16, 16)])


def _deg_pass(dstp, ones, zeros1):
    return pl.kernel(
        _deg_body,
        out_type=jax.ShapeDtypeStruct((_NPAD, 128), jnp.float32),
        mesh=_mesh(),
        scratch_types=[
            pltpu.VMEM((_EPT_CH, _CHUNK), jnp.int32),
            pltpu.VMEM((_CHUNK,), jnp.float32),
            pltpu.VMEM((_RPT,), jnp.float32),
            pltpu.VMEM((_RPT, 16), jnp.float32),
            pltpu.VMEM_SHARED((_NPAD,), jnp.float32),
            pltpu.SemaphoreType.DMA,
        ],
        compiler_params=_sc_params_nolayout,
    )(dstp, ones, zeros1)


def _edge_body(d, src_hbm, dst_hbm, gp_hbm, zeros_hbm, gtab_hbm, out_hbm,
               sidx, didx, rows, gbuf, acc_sh, *sems):
    c = lax.axis_index("core")
    s = lax.axis_index("subcore")
    wid = c * _NS + s
    pltpu.sync_copy(zeros_hbm.at[pl.ds(s * _RPT, _RPT), :],
                    acc_sh.at[pl.ds(s * _RPT, _RPT), :])
    pltpu.sync_copy(gp_hbm.at[pl.ds(s * _RPT, _RPT), pl.ds(0, d)], gbuf)
    pltpu.sync_copy(gbuf, gtab_hbm.at[c, pl.ds(s * _RPT, _RPT), :])
    pltpu.sync_copy(src_hbm.at[wid], sidx)
    pltpu.sync_copy(dst_hbm.at[wid], didx)
    plsc.subcore_barrier()
    gtab = gtab_hbm.at[c]

    def g_start(j, k):
        pltpu.async_copy(gtab.at[sidx.at[j]], rows.at[k], sems[k])

    def g_wait(k):
        pltpu.make_async_copy(gtab.at[sidx.at[0]], rows.at[k], sems[k]).wait()

    def s_start(j, k):
        pltpu.async_copy(rows.at[k], acc_sh.at[didx.at[j]], sems[k], add=True)

    def s_wait(k):
        pltpu.make_async_copy(rows.at[k], acc_sh.at[didx.at[0]],
                              sems[k]).wait()

    for k in range(_NBUF):
        g_start(k, k)

    @pl.loop(0, _EPT_CH // _NBUF - 1)
    def _(q):
        j = q * _NBUF
        for k in range(_NBUF):
            g_wait(k)
            s_start(j + k, k)
        for k in range(_NBUF):
            s_wait(k)
            g_start(j + _NBUF + k, k)

    jl = _EPT_CH - _NBUF
    for k in range(_NBUF):
        g_wait(k)
        s_start(jl + k, k)
    for k in range(_NBUF):
        s_wait(k)

    plsc.subcore_barrier()
    loff = c * (16 if d == _H else 64)
    pltpu.sync_copy(acc_sh.at[pl.ds(s * _RPT, _RPT), :],
                    out_hbm.at[pl.ds(s * _RPT, _RPT), pl.ds(loff, d)])


def _edge_pass(d, srcp, dstp, gp, zeros):
    _, accb = pl.kernel(
        functools.partial(_edge_body, d),
        out_type=[
            jax.ShapeDtypeStruct((_NC, _NPAD, d), jnp.float32),
            jax.ShapeDtypeStruct((_NPAD, 128), jnp.float32),
        ],
        mesh=_mesh(),
        scratch_types=[
            pltpu.VMEM((_EPT_CH, _CHUNK), jnp.int32),
            pltpu.VMEM((_EPT_CH, _CHUNK), jnp.int32),
            pltpu.VMEM((_NBUF, _CHUNK, d), jnp.float32),
            pltpu.VMEM((_RPT, d), jnp.float32),
            pltpu.VMEM_SHARED((_NPAD, d), jnp.float32),
        ] + [pltpu.SemaphoreType.DMA] * _NBUF,
        compiler_params=_sc_params,
    )(srcp, dstp, gp, zeros)
    return accb


def _b1_body(degb_ref, x_ref, w1_ref, g1p_ref, dis_ref):
    deg = degb_ref[:, 0:1] + degb_ref[:, 16:17] + 1.0
    dis = lax.rsqrt(deg)
    hp = jnp.dot(x_ref[...], w1_ref[...], preferred_element_type=jnp.float32)
    g1p_ref[pl.ds(0, _N), :] = hp * dis[0:_N, :]
    g1p_ref[pl.ds(_N, _NPAD - _N), :] = jnp.zeros((_NPAD - _N, 128),
                                                  jnp.float32)
    dis_ref[...] = dis[0:_N, :]


def _b2_body(acc_ref, g1_ref, dis_ref, w2_ref, b1_ref, g2p_ref):
    dis = dis_ref[...]
    out1 = ((acc_ref[pl.ds(0, _N), 0:_H] + acc_ref[pl.ds(0, _N), 16:32]
             + g1_ref[pl.ds(0, _N), 0:_H]) * dis + b1_ref[...])
    h2p = jnp.dot(out1, w2_ref[...], preferred_element_type=jnp.float32)
    g2p_ref[pl.ds(0, _N), :] = h2p * dis
    g2p_ref[pl.ds(_N, _NPAD - _N), :] = jnp.zeros((_NPAD - _N, 128),
                                                  jnp.float32)


def _b3_body(acc_ref, g2_ref, dis_ref, b2_ref, out_ref):
    o48 = ((acc_ref[pl.ds(0, _N), 0:_D2] + acc_ref[pl.ds(0, _N), 64:64 + _D2]
            + g2_ref[pl.ds(0, _N), 0:_D2]) * dis_ref[...] + b2_ref[...])
    out_ref[...] = o48[:, 0:_C]


def kernel(x, edge_index, W1, b1, W2, b2):
    src, dst = edge_index[0], edge_index[1]
    pad = _TRASH + (jnp.arange(_EPAD - _E, dtype=jnp.int32) % _CHUNK)
    srcp = jnp.concatenate([src, pad]).reshape(_NW, _EPT_CH, _CHUNK)
    dstp = jnp.concatenate([dst, pad]).reshape(_NW, _EPT_CH, _CHUNK)
    dstp = lax.optimization_barrier(dstp)
    srcp = lax.optimization_barrier(srcp)
    ones = jnp.ones((_CHUNK,), jnp.float32)
    zeros1 = jnp.zeros((_NPAD,), jnp.float32)
    zeros_h = jnp.zeros((_NPAD, _H), jnp.float32)
    zeros_d2 = jnp.zeros((_NPAD, _D2), jnp.float32)
    w1p = jnp.pad(W1, ((0, 0), (0, 128 - _H)))
    w2p = jnp.pad(W2, ((0, 0), (0, 128 - _C)))
    b1r = b1.reshape(1, _H)
    b2p = jnp.pad(b2, (0, _D2 - _C)).reshape(1, _D2)

    degb = _deg_pass(dstp, ones, zeros1)

    g1p, dis = pl.pallas_call(
        _b1_body,
        out_shape=[
            jax.ShapeDtypeStruct((_NPAD, 128), jnp.float32),
            jax.ShapeDtypeStruct((_N, 1), jnp.float32),
        ],
    )(degb, x, w1p)

    acc1 = _edge_pass(_H, srcp, dstp, g1p, zeros_h)

    g2p = pl.pallas_call(
        _b2_body,
        out_shape=jax.ShapeDtypeStruct((_NPAD, 128), jnp.float32),
    )(acc1, g1p, dis, w2p, b1r)

    acc2 = _edge_pass(_D2, srcp, dstp, g2p, zeros_d2)

    return pl.pallas_call(
        _b3_body,
        out_shape=jax.ShapeDtypeStruct((_N, _C), jnp.float32),
    )(acc2, g2p, dis, b2p)
